# Initial kernel scaffold; baseline (speedup 1.0000x reference)
import jax
import jax.numpy as jnp
from jax.experimental import pallas as pl


def kernel(x, adjs, W_in, W_layers, W_ih, W_hh, b_ih, b_hh, W_out):
    raise NotImplementedError("write your pallas kernel here")


# speedup vs baseline: 6.3195x; 6.3195x over previous
import functools

import jax
import jax.numpy as jnp
from jax import lax
from jax.experimental import pallas as pl
from jax.experimental.pallas import tpu as pltpu
from jax.experimental.pallas import tpu_sc as plsc


@functools.partial(jax.jit, static_argnums=(4, 5, 6))
def _segment_sum_sc(h, src2d, dst2d, zrows, NP, H, E):
    NW = 32
    K = src2d.shape[2]
    CH = (E // NW) // K
    NT = 16
    RPT = NP // NT

    mesh = plsc.VectorSubcoreMesh(core_axis_name="c", subcore_axis_name="s")

    @functools.partial(
        pl.kernel,
        out_type=jax.ShapeDtypeStruct((2, NP, H), jnp.float32),
        mesh=mesh,
        scratch_types=[
            pltpu.VMEM((CH, K), jnp.int32),
            pltpu.VMEM((CH, K), jnp.int32),
            pltpu.VMEM((K, H), jnp.float32),
            pltpu.VMEM_SHARED((NP, H), jnp.float32),
            pltpu.SemaphoreType.DMA,
        ],
    )
    def seg_sum(h_hbm, src_hbm, dst_hbm, z_hbm, out_hbm, src_v, dst_v, rows,
                acc, gsem):
        c = lax.axis_index("c")
        s = lax.axis_index("s")
        wid = s * 2 + c

        pltpu.sync_copy(z_hbm.at[pl.ds(s * RPT, RPT)],
                        acc.at[pl.ds(s * RPT, RPT)])

        pltpu.sync_copy(src_hbm.at[wid], src_v)
        pltpu.sync_copy(dst_hbm.at[wid], dst_v)

        plsc.subcore_barrier()

        @pl.loop(0, CH)
        def _(j):
            pltpu.async_copy(h_hbm.at[src_v.at[j]], rows, gsem).wait()
            pltpu.sync_copy(rows, acc.at[dst_v.at[j]], add=True)

        plsc.subcore_barrier()

        pltpu.sync_copy(acc.at[pl.ds(s * RPT, RPT)],
                        out_hbm.at[c, pl.ds(s * RPT, RPT)])

    return seg_sum(h, src2d, dst2d, zrows)


def _matmul_in(x, W_in, BN=2000):
    N, F = x.shape
    H = W_in.shape[1]

    def body(x_ref, w_ref, o_ref):
        o_ref[...] = jnp.dot(x_ref[...], w_ref[...],
                             preferred_element_type=jnp.float32)

    return pl.pallas_call(
        body,
        grid=(N // BN,),
        in_specs=[
            pl.BlockSpec((BN, F), lambda i: (i, 0)),
            pl.BlockSpec((F, H), lambda i: (0, 0)),
        ],
        out_specs=pl.BlockSpec((BN, H), lambda i: (i, 0)),
        out_shape=jax.ShapeDtypeStruct((N, H), jnp.float32),
    )(x, W_in)


def _gru_layer(p, h, W_ihT, W_hhT, b_ih2, b_hh2, W_out=None,
               BN=2000):
    N, H = h.shape
    final = W_out is not None
    CO = W_out.shape[1] if final else H

    def body(p0_ref, p1_ref, h_ref, wih_ref, whh_ref, bi_ref, bh_ref,
             *rest):
        if final:
            wout_ref, o_ref = rest
        else:
            (o_ref,) = rest
        hv = h_ref[...]
        agg = p0_ref[0] + p1_ref[0]
        gi = jnp.dot(agg, wih_ref[...],
                     preferred_element_type=jnp.float32) + bi_ref[...]
        gh = jnp.dot(hv, whh_ref[...],
                     preferred_element_type=jnp.float32) + bh_ref[...]
        r = jax.nn.sigmoid(gi[:, :H] + gh[:, :H])
        z = jax.nn.sigmoid(gi[:, H:2 * H] + gh[:, H:2 * H])
        n = jnp.tanh(gi[:, 2 * H:] + r * gh[:, 2 * H:])
        hn = (1.0 - z) * n + z * hv
        if final:
            o_ref[...] = jnp.dot(hn, wout_ref[...],
                                 preferred_element_type=jnp.float32)
        else:
            o_ref[...] = hn

    in_specs = [
        pl.BlockSpec((1, BN, H), lambda i: (0, i, 0)),
        pl.BlockSpec((1, BN, H), lambda i: (1, i, 0)),
        pl.BlockSpec((BN, H), lambda i: (i, 0)),
        pl.BlockSpec((H, 3 * H), lambda i: (0, 0)),
        pl.BlockSpec((H, 3 * H), lambda i: (0, 0)),
        pl.BlockSpec((1, 3 * H), lambda i: (0, 0)),
        pl.BlockSpec((1, 3 * H), lambda i: (0, 0)),
    ]
    args = [p, p, h, W_ihT, W_hhT, b_ih2, b_hh2]
    if final:
        in_specs.append(pl.BlockSpec((H, CO), lambda i: (0, 0)))
        args.append(W_out)

    return pl.pallas_call(
        body,
        grid=(N // BN,),
        in_specs=in_specs,
        out_specs=pl.BlockSpec((BN, CO), lambda i: (i, 0)),
        out_shape=jax.ShapeDtypeStruct((N, CO), jnp.float32),
    )(*args)


def kernel(x, adjs, W_in, W_layers, W_ih, W_hh, b_ih, b_hh, W_out):
    N, F = x.shape
    H = W_in.shape[1]
    E = adjs.shape[1]
    L = W_layers.shape[0]

    NW, K = 32, 80
    CH = (E // NW) // K
    NP = 16 * 632
    src2d = adjs[0].reshape(NW, CH, K)
    dst2d = adjs[1].reshape(NW, CH, K)
    zrows = jnp.zeros((NP, H), jnp.float32)

    W_ihT = W_ih.T.astype(jnp.float32)
    W_hhT = W_hh.T.astype(jnp.float32)
    b_ih2 = b_ih.reshape(1, -1)
    b_hh2 = b_hh.reshape(1, -1)

    h = _matmul_in(x, W_in)
    out = None
    for l in range(L):
        m = _matmul_in(h, W_layers[l])
        p = _segment_sum_sc(m, src2d, dst2d, zrows, NP, H, E)
        res = _gru_layer(p, h, W_ihT, W_hhT,
                         b_ih2, b_hh2, W_out if l == L - 1 else None)
        if l == L - 1:
            out = res
        else:
            h = res
    return out
